# baseline (speedup 1.0000x reference)
import functools

import numpy as np
import jax
import jax.numpy as jnp
from jax import lax
from jax.experimental import pallas as pl
from jax.experimental.pallas import tpu as pltpu
from jax.experimental.pallas import tpu_sc as plsc

D_MODEL = 768
MAX_SEQ_LEN = 8192
_SEED_STRIDE = 128


def _pos_encoding(max_len, d_model):
    pos = np.arange(max_len, dtype=np.float32)[:, None]
    _2i = np.arange(0, d_model, 2, dtype=np.float32)
    div = np.power(10000.0, _2i / d_model)
    pe = np.zeros((max_len, d_model), dtype=np.float32)
    pe[:, 0::2] = np.sin(pos / div)
    pe[:, 1::2] = np.cos(pos / div)
    return pe


def _pe_coef(d_model):
    _2i = np.arange(0, d_model, 2, dtype=np.float64)
    omega = 1.0 / np.power(10000.0, _2i / d_model)
    coef = 2.0 * np.cos(omega)
    return np.repeat(coef, 2).astype(np.float32)


_PE = _pos_encoding(MAX_SEQ_LEN, D_MODEL)


@functools.lru_cache(maxsize=None)
def _pe_seeds_device(S):
    seeds = _PE[:S].reshape(S // _SEED_STRIDE, _SEED_STRIDE, D_MODEL)[:, :2]
    return (jnp.asarray(seeds.reshape(-1, D_MODEL)),
            jnp.asarray(_pe_coef(D_MODEL)[None, :]))


_NC = 2
_NS = 16
_NW = _NC * _NS
_CHUNK = 16


@functools.lru_cache(maxsize=None)
def _make_kernel(B, S, D):
    total = B * S
    per_pos = S // _NW
    assert per_pos * _NW == S and per_pos % (2 * _CHUNK) == 0
    assert per_pos % _SEED_STRIDE == 0 and _SEED_STRIDE % _CHUNK == 0
    n_chunks = per_pos // _CHUNK
    n_seed = per_pos // _SEED_STRIDE
    cpg = _SEED_STRIDE // _CHUNK
    n_col = D // 16
    mesh = plsc.VectorSubcoreMesh(core_axis_name="c", subcore_axis_name="s")

    row_t = pltpu.VMEM((_CHUNK, D), jnp.float32)
    sem_t = pltpu.SemaphoreType.DMA

    @functools.partial(
        pl.kernel,
        mesh=mesh,
        out_type=jax.ShapeDtypeStruct((total, D), jnp.float32),
        scratch_types=(
            [pltpu.VMEM((B, per_pos), jnp.int32),
             pltpu.VMEM((2 * n_seed, D), jnp.float32),
             pltpu.VMEM((1, D), jnp.float32),
             pltpu.VMEM((2, D), jnp.float32)]
            + [row_t] * (2 * B)
            + [sem_t] * (4 * B + 1)
        ),
    )
    def k(idx_hbm, table_hbm, seeds_hbm, coef_hbm, out_hbm,
          idx_v, seeds_v, coef_v, carry_v, *bufs_and_sems):
        rows = (bufs_and_sems[:B], bufs_and_sems[B:2 * B])
        gsem = (bufs_and_sems[2 * B:3 * B], bufs_and_sems[3 * B:4 * B])
        wsem = (bufs_and_sems[4 * B:5 * B], bufs_and_sems[5 * B:6 * B])
        psem = bufs_and_sems[6 * B]

        wid = lax.axis_index("s") * _NC + lax.axis_index("c")
        pos0 = wid * per_pos
        for b in range(B):
            pltpu.async_copy(idx_hbm.at[b, pl.ds(pos0, per_pos)],
                             idx_v.at[b], wsem[0][b])
        for b in range(B):
            pltpu.make_async_copy(idx_hbm.at[b, pl.ds(pos0, per_pos)],
                                  idx_v.at[b], wsem[0][b]).wait()

        def g_start(pc, b, st):
            pltpu.async_copy(
                table_hbm.at[idx_v.at[b, pl.ds(pc * _CHUNK, _CHUNK)]],
                rows[st][b], gsem[st][b])

        def g_wait(pc, b, st):
            pltpu.make_async_copy(
                table_hbm.at[idx_v.at[b, pl.ds(pc * _CHUNK, _CHUNK)]],
                rows[st][b], gsem[st][b]).wait()

        def w_start(pc, b, st):
            pltpu.async_copy(
                rows[st][b],
                out_hbm.at[pl.ds(b * S + pos0 + pc * _CHUNK, _CHUNK)],
                wsem[st][b])

        def w_wait(b, st):
            pltpu.make_async_copy(rows[st][b], out_hbm.at[pl.ds(0, _CHUNK)],
                                  wsem[st][b]).wait()

        for b in range(B):
            g_start(0, b, 0)
        pltpu.async_copy(seeds_hbm.at[pl.ds(wid * 2 * n_seed, 2 * n_seed)],
                         seeds_v, psem)
        pltpu.async_copy(coef_hbm, coef_v, psem)
        pltpu.make_async_copy(
            seeds_hbm.at[pl.ds(0, 2 * n_seed)], seeds_v, psem).wait()
        pltpu.make_async_copy(coef_hbm, coef_v, psem).wait()

        @pl.loop(0, n_seed)
        def _(sg):
            for dj in range(cpg):
                pc = sg * cpg + dj
                st = dj % 2
                nst = 1 - st
                @pl.when(pc + 1 < n_chunks)
                def _():
                    for b in range(B):
                        if dj == 0:

                            @pl.when(sg > 0)
                            def _():
                                w_wait(b, nst)
                        else:
                            w_wait(b, nst)
                        g_start(pc + 1, b, nst)

                for b in range(B):
                    g_wait(pc, b, st)

                cur = rows[st]
                last_in_group = dj == cpg - 1

                @pl.loop(0, n_col)
                def _(c):
                    cs = pl.ds(c * 16, 16)
                    cf = coef_v[0, cs]
                    if dj == 0:
                        pm1 = seeds_v[2 * sg, cs]
                        p = seeds_v[2 * sg + 1, cs]
                    else:
                        pm1 = carry_v[0, cs]
                        p = carry_v[1, cs]

                    for b in range(B):
                        plsc.addupdate(cur[b].at[0, cs], pm1)
                    for b in range(B):
                        plsc.addupdate(cur[b].at[1, cs], p)
                    for r in range(2, _CHUNK):
                        pm1, p = p, cf * p - pm1
                        for b in range(B):
                            plsc.addupdate(cur[b].at[r, cs], p)
                    if not last_in_group:
                        pm1, p = p, cf * p - pm1
                        pm1, p = p, cf * p - pm1
                        carry_v[0, cs] = pm1
                        carry_v[1, cs] = p

                for b in range(B):
                    w_start(pc, b, st)

        for st in range(2):
            for b in range(B):
                w_wait(b, st)

    return k


@jax.jit
def _run(x, table, seeds, coef):
    B, S = x.shape
    D = table.shape[1]
    out = _make_kernel(B, S, D)(x, table, seeds, coef)
    return out.reshape(B, S, D)


def kernel(x, table):
    seeds, coef = _pe_seeds_device(x.shape[1])
    return _run(x, table, seeds, coef)


# speedup vs baseline: 1.0255x; 1.0255x over previous
import functools

import numpy as np
import jax
import jax.numpy as jnp
from jax import lax
from jax.experimental import pallas as pl
from jax.experimental.pallas import tpu as pltpu
from jax.experimental.pallas import tpu_sc as plsc

D_MODEL = 768
MAX_SEQ_LEN = 8192
_SEED_STRIDE = 64


def _pos_encoding(max_len, d_model):
    pos = np.arange(max_len, dtype=np.float32)[:, None]
    _2i = np.arange(0, d_model, 2, dtype=np.float32)
    div = np.power(10000.0, _2i / d_model)
    pe = np.zeros((max_len, d_model), dtype=np.float32)
    pe[:, 0::2] = np.sin(pos / div)
    pe[:, 1::2] = np.cos(pos / div)
    return pe


def _pe_coef(d_model):
    _2i = np.arange(0, d_model, 2, dtype=np.float64)
    omega = 1.0 / np.power(10000.0, _2i / d_model)
    coef = 2.0 * np.cos(omega)
    return np.repeat(coef, 2).astype(np.float32)


_PE = _pos_encoding(MAX_SEQ_LEN, D_MODEL)


@functools.lru_cache(maxsize=None)
def _pe_seeds_device(S):
    seeds = _PE[:S].reshape(S // _SEED_STRIDE, _SEED_STRIDE, D_MODEL)[:, :2]
    return (jnp.asarray(seeds.reshape(-1, D_MODEL)),
            jnp.asarray(_pe_coef(D_MODEL)[None, :]))


_NC = 2
_NS = 16
_NW = _NC * _NS
_CHUNK = 16


@functools.lru_cache(maxsize=None)
def _make_kernel(B, S, D):
    total = B * S
    per_pos = S // _NW
    assert per_pos * _NW == S and per_pos % (2 * _CHUNK) == 0
    assert per_pos % _SEED_STRIDE == 0 and _SEED_STRIDE % _CHUNK == 0
    n_chunks = per_pos // _CHUNK
    n_seed = per_pos // _SEED_STRIDE
    cpg = _SEED_STRIDE // _CHUNK
    n_col = D // 16
    mesh = plsc.VectorSubcoreMesh(core_axis_name="c", subcore_axis_name="s")

    row_t = pltpu.VMEM((_CHUNK, D), jnp.float32)
    sem_t = pltpu.SemaphoreType.DMA

    @functools.partial(
        pl.kernel,
        mesh=mesh,
        out_type=jax.ShapeDtypeStruct((total, D), jnp.float32),
        scratch_types=(
            [pltpu.VMEM((B, per_pos), jnp.int32),
             pltpu.VMEM((2 * n_seed, D), jnp.float32),
             pltpu.VMEM((1, D), jnp.float32),
             pltpu.VMEM((2, D), jnp.float32)]
            + [row_t] * (2 * B)
            + [sem_t] * (4 * B + 1)
        ),
    )
    def k(idx_hbm, table_hbm, seeds_hbm, coef_hbm, out_hbm,
          idx_v, seeds_v, coef_v, carry_v, *bufs_and_sems):
        rows = (bufs_and_sems[:B], bufs_and_sems[B:2 * B])
        gsem = (bufs_and_sems[2 * B:3 * B], bufs_and_sems[3 * B:4 * B])
        wsem = (bufs_and_sems[4 * B:5 * B], bufs_and_sems[5 * B:6 * B])
        psem = bufs_and_sems[6 * B]

        wid = lax.axis_index("s") * _NC + lax.axis_index("c")
        pos0 = wid * per_pos
        for b in range(B):
            pltpu.async_copy(idx_hbm.at[b, pl.ds(pos0, per_pos)],
                             idx_v.at[b], wsem[0][b])
        for b in range(B):
            pltpu.make_async_copy(idx_hbm.at[b, pl.ds(pos0, per_pos)],
                                  idx_v.at[b], wsem[0][b]).wait()

        def g_start(pc, b, st):
            pltpu.async_copy(
                table_hbm.at[idx_v.at[b, pl.ds(pc * _CHUNK, _CHUNK)]],
                rows[st][b], gsem[st][b])

        def g_wait(pc, b, st):
            pltpu.make_async_copy(
                table_hbm.at[idx_v.at[b, pl.ds(pc * _CHUNK, _CHUNK)]],
                rows[st][b], gsem[st][b]).wait()

        def w_start(pc, b, st):
            pltpu.async_copy(
                rows[st][b],
                out_hbm.at[pl.ds(b * S + pos0 + pc * _CHUNK, _CHUNK)],
                wsem[st][b])

        def w_wait(b, st):
            pltpu.make_async_copy(rows[st][b], out_hbm.at[pl.ds(0, _CHUNK)],
                                  wsem[st][b]).wait()

        for b in range(B):
            g_start(0, b, 0)
        pltpu.async_copy(seeds_hbm.at[pl.ds(wid * 2 * n_seed, 2 * n_seed)],
                         seeds_v, psem)
        pltpu.async_copy(coef_hbm, coef_v, psem)
        pltpu.make_async_copy(
            seeds_hbm.at[pl.ds(0, 2 * n_seed)], seeds_v, psem).wait()
        pltpu.make_async_copy(coef_hbm, coef_v, psem).wait()

        @pl.loop(0, n_seed)
        def _(sg):
            for dj in range(cpg):
                pc = sg * cpg + dj
                st = dj % 2
                nst = 1 - st
                @pl.when(pc + 1 < n_chunks)
                def _():
                    for b in range(B):
                        if dj == 0:

                            @pl.when(sg > 0)
                            def _():
                                w_wait(b, nst)
                        else:
                            w_wait(b, nst)
                        g_start(pc + 1, b, nst)

                for b in range(B):
                    g_wait(pc, b, st)

                cur = rows[st]
                last_in_group = dj == cpg - 1

                @pl.loop(0, n_col)
                def _(c):
                    cs = pl.ds(c * 16, 16)
                    cf = coef_v[0, cs]
                    if dj == 0:
                        pm1 = seeds_v[2 * sg, cs]
                        p = seeds_v[2 * sg + 1, cs]
                    else:
                        pm1 = carry_v[0, cs]
                        p = carry_v[1, cs]

                    for b in range(B):
                        plsc.addupdate(cur[b].at[0, cs], pm1)
                    for b in range(B):
                        plsc.addupdate(cur[b].at[1, cs], p)
                    for r in range(2, _CHUNK):
                        pm1, p = p, cf * p - pm1
                        for b in range(B):
                            plsc.addupdate(cur[b].at[r, cs], p)
                    if not last_in_group:
                        pm1, p = p, cf * p - pm1
                        pm1, p = p, cf * p - pm1
                        carry_v[0, cs] = pm1
                        carry_v[1, cs] = p

                for b in range(B):
                    w_start(pc, b, st)

        for st in range(2):
            for b in range(B):
                w_wait(b, st)

    return k


@jax.jit
def _run(x, table, seeds, coef):
    B, S = x.shape
    D = table.shape[1]
    out = _make_kernel(B, S, D)(x, table, seeds, coef)
    return out.reshape(B, S, D)


def kernel(x, table):
    seeds, coef = _pe_seeds_device(x.shape[1])
    return _run(x, table, seeds, coef)
